# Initial kernel scaffold; baseline (speedup 1.0000x reference)
import jax
import jax.numpy as jnp
from jax.experimental import pallas as pl


def kernel(x, edge_index, W1, b1, W2, b2):
    raise NotImplementedError("write your pallas kernel here")


# speedup vs baseline: 15.1483x; 15.1483x over previous
import functools

import jax
import jax.numpy as jnp
from jax import lax
from jax.experimental import pallas as pl
from jax.experimental.pallas import tpu as pltpu
from jax.experimental.pallas import tpu_sc as plsc

N = 10000
E = 320000
D = 128

NC = 2
NS = 16
NW = NC * NS
L = 16

K = 128
CH = 79
T = K * CH
EP = NW * T
PAD = EP - E

R = 10240
RT = R // NS
RC = RT // K

BM = 1000
GRID = N // BM

_mesh = plsc.VectorSubcoreMesh(
    core_axis_name="c", subcore_axis_name="s", num_cores=NC, num_subcores=NS
)


@functools.partial(
    pl.kernel,
    out_type=jax.ShapeDtypeStruct((NC * R, D), jnp.float32),
    mesh=_mesh,
    scratch_types=[
        pltpu.VMEM((K,), jnp.int32),
        pltpu.VMEM((K, D), jnp.float32),
        pltpu.MemorySpace.VMEM_SHARED((R, D), jnp.float32),
    ],
)
def _deg_kernel(dst_hbm, out_hbm, idx_v, ones_v, acc_sh):
    cid = lax.axis_index("c")
    sid = lax.axis_index("s")
    wid = cid * NS + sid

    def zfill(i, _):
        for j in range(D // L):
            ones_v[i, pl.ds(j * L, L)] = jnp.zeros((L,), jnp.float32)
        return 0

    lax.fori_loop(0, K, zfill, 0)
    for j in range(RC):
        pltpu.sync_copy(ones_v, acc_sh.at[pl.ds(sid * RT + j * K, K)])
    plsc.subcore_barrier()

    def ofill(i, _):
        for j in range(D // L):
            ones_v[i, pl.ds(j * L, L)] = jnp.ones((L,), jnp.float32)
        return 0

    lax.fori_loop(0, K, ofill, 0)

    def body(ch, _):
        base = wid * T + ch * K
        pltpu.sync_copy(dst_hbm.at[pl.ds(base, K)], idx_v)
        pltpu.sync_copy(ones_v, acc_sh.at[idx_v], add=True)
        return 0

    lax.fori_loop(0, CH, body, 0)
    plsc.subcore_barrier()
    for j in range(RC):
        pltpu.sync_copy(acc_sh.at[pl.ds(sid * RT + j * K, K)], ones_v)
        pltpu.sync_copy(
            ones_v, out_hbm.at[pl.ds(cid * R + sid * RT + j * K, K)]
        )


@functools.partial(
    pl.kernel,
    out_type=jax.ShapeDtypeStruct((NC * R, D), jnp.float32),
    mesh=_mesh,
    scratch_types=[
        pltpu.VMEM((K,), jnp.int32),
        pltpu.VMEM((K,), jnp.int32),
        pltpu.VMEM((K, D), jnp.float32),
        pltpu.MemorySpace.VMEM_SHARED((R, D), jnp.float32),
        pltpu.SemaphoreType.DMA,
    ],
)
def _scatter_kernel(g_hbm, src_hbm, dst_hbm, out_hbm, src_v, dst_v, rows_v,
                    acc_sh, sem):
    cid = lax.axis_index("c")
    sid = lax.axis_index("s")
    wid = cid * NS + sid

    def zfill(i, _):
        for j in range(D // L):
            rows_v[i, pl.ds(j * L, L)] = jnp.zeros((L,), jnp.float32)
        return 0

    lax.fori_loop(0, K, zfill, 0)
    for j in range(RC):
        pltpu.sync_copy(rows_v, acc_sh.at[pl.ds(sid * RT + j * K, K)])
    plsc.subcore_barrier()

    def body(ch, _):
        base = wid * T + ch * K
        pltpu.sync_copy(src_hbm.at[pl.ds(base, K)], src_v)
        pltpu.sync_copy(dst_hbm.at[pl.ds(base, K)], dst_v)
        pltpu.async_copy(g_hbm.at[src_v], rows_v, sem).wait()
        pltpu.sync_copy(rows_v, acc_sh.at[dst_v], add=True)
        return 0

    lax.fori_loop(0, CH, body, 0)
    plsc.subcore_barrier()
    for j in range(RC):
        pltpu.sync_copy(acc_sh.at[pl.ds(sid * RT + j * K, K)], rows_v)
        pltpu.sync_copy(
            rows_v, out_hbm.at[pl.ds(cid * R + sid * RT + j * K, K)]
        )


def _dinv_from(dp):
    deg = dp[0, :, 0] + dp[1, :, 0] + 1.0
    return lax.rsqrt(deg)


def _mm_scale_body(x_ref, w_ref, dp_ref, g_ref):
    dinv = _dinv_from(dp_ref[...])
    h = jnp.dot(x_ref[...], w_ref[...], preferred_element_type=jnp.float32)
    g_ref[...] = h * dinv[:, None]


def _comb_mm_body(p_ref, g_ref, dp_ref, b_ref, w_ref, o_ref):
    dinv = _dinv_from(dp_ref[...])
    p = p_ref[...]
    s = (p[0] + p[1] + g_ref[...]) * dinv[:, None] + b_ref[...][0:1, :]
    a = jnp.maximum(s, 0.0)
    o_ref[...] = (
        jnp.dot(a, w_ref[...], preferred_element_type=jnp.float32)
        * dinv[:, None]
    )


def _final_body(p_ref, g_ref, dp_ref, b_ref, o_ref):
    dinv = _dinv_from(dp_ref[...])
    p = p_ref[...]
    o_ref[...] = (p[0] + p[1] + g_ref[...]) * dinv[:, None] + b_ref[...][0:1, :]


_row_spec = pl.BlockSpec((BM, D), lambda i: (i, 0))
_w_spec = pl.BlockSpec((D, D), lambda i: (0, 0))
_p_spec = pl.BlockSpec((NC, BM, D), lambda i: (0, i, 0))
_b_spec = pl.BlockSpec((8, D), lambda i: (0, 0))
_row_out = jax.ShapeDtypeStruct((N, D), jnp.float32)

_mm_scale = pl.pallas_call(
    _mm_scale_body,
    grid=(GRID,),
    in_specs=[_row_spec, _w_spec, _p_spec],
    out_specs=_row_spec,
    out_shape=_row_out,
)

_comb_mm = pl.pallas_call(
    _comb_mm_body,
    grid=(GRID,),
    in_specs=[_p_spec, _row_spec, _p_spec, _b_spec, _w_spec],
    out_specs=_row_spec,
    out_shape=_row_out,
)

_final = pl.pallas_call(
    _final_body,
    grid=(GRID,),
    in_specs=[_p_spec, _row_spec, _p_spec, _b_spec],
    out_specs=_row_spec,
    out_shape=_row_out,
)


def kernel(x, edge_index, W1, b1, W2, b2):
    ei = edge_index.astype(jnp.int32)
    src, dst = ei[0], ei[1]
    ar = jnp.arange(PAD, dtype=jnp.int32)
    src_p = jnp.concatenate([src, (ar * 37) % N])
    dst_p = jnp.concatenate([dst, N + ar % (R - N)])
    b1r = jnp.broadcast_to(b1.reshape(1, D), (8, D))
    b2r = jnp.broadcast_to(b2.reshape(1, D), (8, D))

    degp = _deg_kernel(dst_p).reshape(NC, R, D)

    g1 = _mm_scale(x, W1, degp)
    p1 = _scatter_kernel(g1, src_p, dst_p).reshape(NC, R, D)
    g2 = _comb_mm(p1, g1, degp, b1r, W2)
    p2 = _scatter_kernel(g2, src_p, dst_p).reshape(NC, R, D)
    return _final(p2, g2, degp, b2r)


# speedup vs baseline: 27.6595x; 1.8259x over previous
import functools

import jax
import jax.numpy as jnp
from jax import lax
from jax.experimental import pallas as pl
from jax.experimental.pallas import tpu as pltpu
from jax.experimental.pallas import tpu_sc as plsc

N = 10000
E = 320000
D = 128

NC = 2
NS = 16
NW = NC * NS
L = 16

K = 128
CH = 80
T = K * CH
EP = NW * T
PAD = EP - E

R = 10240
RT = R // NS
RC = RT // K

BM = 1000
GRID = N // BM

_mesh = plsc.VectorSubcoreMesh(
    core_axis_name="c", subcore_axis_name="s", num_cores=NC, num_subcores=NS
)


@functools.partial(
    pl.kernel,
    out_type=jax.ShapeDtypeStruct((NC * R, D), jnp.float32),
    mesh=_mesh,
    scratch_types=[
        pltpu.VMEM((CH, K), jnp.int32),
        pltpu.VMEM((K, D), jnp.float32),
        pltpu.MemorySpace.VMEM_SHARED((R, D), jnp.float32),
    ],
)
def _deg_kernel(dst_hbm, out_hbm, dv, ones_v, acc_sh):
    cid = lax.axis_index("c")
    sid = lax.axis_index("s")
    wid = cid * NS + sid

    def zfill(i, _):
        for j in range(D // L):
            ones_v[i, pl.ds(j * L, L)] = jnp.zeros((L,), jnp.float32)
        return 0

    lax.fori_loop(0, K, zfill, 0)
    for j in range(RC):
        pltpu.sync_copy(ones_v, acc_sh.at[pl.ds(sid * RT + j * K, K)])
    plsc.subcore_barrier()

    def ofill(i, _):
        for j in range(D // L):
            ones_v[i, pl.ds(j * L, L)] = jnp.ones((L,), jnp.float32)
        return 0

    lax.fori_loop(0, K, ofill, 0)
    pltpu.sync_copy(dst_hbm.at[pl.ds(wid * CH, CH)], dv)

    def body(ch, _):
        pltpu.sync_copy(ones_v, acc_sh.at[dv.at[ch]], add=True)
        return 0

    lax.fori_loop(0, CH, body, 0)
    plsc.subcore_barrier()
    for j in range(RC):
        pltpu.sync_copy(acc_sh.at[pl.ds(sid * RT + j * K, K)], ones_v)
        pltpu.sync_copy(
            ones_v, out_hbm.at[pl.ds(cid * R + sid * RT + j * K, K)]
        )


@functools.partial(
    pl.kernel,
    out_type=jax.ShapeDtypeStruct((NC * R, D), jnp.float32),
    mesh=_mesh,
    scratch_types=[
        pltpu.VMEM((CH, K), jnp.int32),
        pltpu.VMEM((K,), jnp.int32),
        pltpu.VMEM((K,), jnp.int32),
        pltpu.VMEM((K, D), jnp.float32),
        pltpu.VMEM((K, D), jnp.float32),
        pltpu.MemorySpace.VMEM_SHARED((R, D), jnp.float32),
        pltpu.SemaphoreType.DMA,
        pltpu.SemaphoreType.DMA,
    ],
)
def _scatter_kernel(g_hbm, src_hbm, dst_hbm, out_hbm, sv, da, db, ra, rb,
                    acc_sh, sem_a, sem_b):
    cid = lax.axis_index("c")
    sid = lax.axis_index("s")
    wid = cid * NS + sid

    def zfill(i, _):
        for j in range(D // L):
            ra[i, pl.ds(j * L, L)] = jnp.zeros((L,), jnp.float32)
        return 0

    lax.fori_loop(0, K, zfill, 0)
    for j in range(RC):
        pltpu.sync_copy(ra, acc_sh.at[pl.ds(sid * RT + j * K, K)])
    plsc.subcore_barrier()

    pltpu.sync_copy(src_hbm.at[pl.ds(wid * CH, CH)], sv)

    pltpu.async_copy(g_hbm.at[sv.at[0]], ra, sem_a)

    def pair(p, _):
        base = wid * T
        pltpu.async_copy(g_hbm.at[sv.at[2 * p + 1]], rb, sem_b)
        pltpu.sync_copy(dst_hbm.at[pl.ds(base + (2 * p) * K, K)], da)
        pltpu.make_async_copy(g_hbm.at[pl.ds(0, K)], ra, sem_a).wait()
        pltpu.sync_copy(ra, acc_sh.at[da], add=True)
        pltpu.async_copy(g_hbm.at[sv.at[2 * p + 2]], ra, sem_a)
        pltpu.sync_copy(dst_hbm.at[pl.ds(base + (2 * p + 1) * K, K)], db)
        pltpu.make_async_copy(g_hbm.at[pl.ds(0, K)], rb, sem_b).wait()
        pltpu.sync_copy(rb, acc_sh.at[db], add=True)
        return 0

    lax.fori_loop(0, (CH - 2) // 2, pair, 0)
    pltpu.sync_copy(dst_hbm.at[pl.ds(wid * T + (CH - 2) * K, K)], da)
    pltpu.make_async_copy(g_hbm.at[pl.ds(0, K)], ra, sem_a).wait()
    pltpu.sync_copy(ra, acc_sh.at[da], add=True)
    pltpu.sync_copy(dst_hbm.at[pl.ds(wid * T + (CH - 1) * K, K)], db)
    pltpu.async_copy(g_hbm.at[sv.at[CH - 1]], rb, sem_b).wait()
    pltpu.sync_copy(rb, acc_sh.at[db], add=True)

    plsc.subcore_barrier()
    for j in range(RC):
        pltpu.sync_copy(acc_sh.at[pl.ds(sid * RT + j * K, K)], ra)
        pltpu.sync_copy(
            ra, out_hbm.at[pl.ds(cid * R + sid * RT + j * K, K)]
        )


def _dinv_from(dp):
    deg = dp[0, :, 0] + dp[1, :, 0] + 1.0
    return lax.rsqrt(deg)


def _mm_scale_body(x_ref, w_ref, dp_ref, g_ref):
    dinv = _dinv_from(dp_ref[...])
    h = jnp.dot(x_ref[...], w_ref[...], preferred_element_type=jnp.float32)
    g_ref[...] = h * dinv[:, None]


def _comb_mm_body(p_ref, g_ref, dp_ref, b_ref, w_ref, o_ref):
    dinv = _dinv_from(dp_ref[...])
    p = p_ref[...]
    s = (p[0] + p[1] + g_ref[...]) * dinv[:, None] + b_ref[...][0:1, :]
    a = jnp.maximum(s, 0.0)
    o_ref[...] = (
        jnp.dot(a, w_ref[...], preferred_element_type=jnp.float32)
        * dinv[:, None]
    )


def _final_body(p_ref, g_ref, dp_ref, b_ref, o_ref):
    dinv = _dinv_from(dp_ref[...])
    p = p_ref[...]
    o_ref[...] = (p[0] + p[1] + g_ref[...]) * dinv[:, None] + b_ref[...][0:1, :]


_row_spec = pl.BlockSpec((BM, D), lambda i: (i, 0))
_w_spec = pl.BlockSpec((D, D), lambda i: (0, 0))
_p_spec = pl.BlockSpec((NC, BM, D), lambda i: (0, i, 0))
_b_spec = pl.BlockSpec((8, D), lambda i: (0, 0))
_row_out = jax.ShapeDtypeStruct((N, D), jnp.float32)

_mm_scale = pl.pallas_call(
    _mm_scale_body,
    grid=(GRID,),
    in_specs=[_row_spec, _w_spec, _p_spec],
    out_specs=_row_spec,
    out_shape=_row_out,
)

_comb_mm = pl.pallas_call(
    _comb_mm_body,
    grid=(GRID,),
    in_specs=[_p_spec, _row_spec, _p_spec, _b_spec, _w_spec],
    out_specs=_row_spec,
    out_shape=_row_out,
)

_final = pl.pallas_call(
    _final_body,
    grid=(GRID,),
    in_specs=[_p_spec, _row_spec, _p_spec, _b_spec],
    out_specs=_row_spec,
    out_shape=_row_out,
)


def kernel(x, edge_index, W1, b1, W2, b2):
    ei = edge_index.astype(jnp.int32)
    src, dst = ei[0], ei[1]
    ar = jnp.arange(PAD, dtype=jnp.int32)
    src_p = jnp.concatenate([src, (ar * 37) % N]).reshape(NW * CH, K)
    dst_p = jnp.concatenate([dst, N + ar % (R - N)])
    dst_p2 = dst_p.reshape(NW * CH, K)
    b1r = jnp.broadcast_to(b1.reshape(1, D), (8, D))
    b2r = jnp.broadcast_to(b2.reshape(1, D), (8, D))

    degp = _deg_kernel(dst_p2).reshape(NC, R, D)

    g1 = _mm_scale(x, W1, degp)
    p1 = _scatter_kernel(g1, src_p, dst_p).reshape(NC, R, D)
    g2 = _comb_mm(p1, g1, degp, b1r, W2)
    p2 = _scatter_kernel(g2, src_p, dst_p).reshape(NC, R, D)
    return _final(p2, g2, degp, b2r)


# speedup vs baseline: 28.5908x; 1.0337x over previous
import functools

import jax
import jax.numpy as jnp
from jax import lax
from jax.experimental import pallas as pl
from jax.experimental.pallas import tpu as pltpu
from jax.experimental.pallas import tpu_sc as plsc

N = 10000
E = 320000
D = 128

NC = 2
NS = 16
NW = NC * NS
L = 16

K = 128
CH = 80
T = K * CH
EP = NW * T
PAD = EP - E

R = 10240
RT = R // NS
RC = RT // K

BM = 1000
GRID = N // BM

_mesh = plsc.VectorSubcoreMesh(
    core_axis_name="c", subcore_axis_name="s", num_cores=NC, num_subcores=NS
)


@functools.partial(
    pl.kernel,
    out_type=jax.ShapeDtypeStruct((NC * R, D), jnp.float32),
    mesh=_mesh,
    scratch_types=[
        pltpu.VMEM((CH, K), jnp.int32),
        pltpu.VMEM((K, D), jnp.float32),
        pltpu.MemorySpace.VMEM_SHARED((R, D), jnp.float32),
    ],
)
def _deg_kernel(dst_hbm, out_hbm, dv, ones_v, acc_sh):
    cid = lax.axis_index("c")
    sid = lax.axis_index("s")
    wid = cid * NS + sid

    def zfill(i, _):
        for j in range(D // L):
            ones_v[i, pl.ds(j * L, L)] = jnp.zeros((L,), jnp.float32)
        return 0

    lax.fori_loop(0, K, zfill, 0)
    for j in range(RC):
        pltpu.sync_copy(ones_v, acc_sh.at[pl.ds(sid * RT + j * K, K)])
    plsc.subcore_barrier()

    def ofill(i, _):
        for j in range(D // L):
            ones_v[i, pl.ds(j * L, L)] = jnp.ones((L,), jnp.float32)
        return 0

    lax.fori_loop(0, K, ofill, 0)
    pltpu.sync_copy(dst_hbm.at[pl.ds(wid * CH, CH)], dv)

    def body(ch, _):
        pltpu.sync_copy(ones_v, acc_sh.at[dv.at[ch]], add=True)
        return 0

    lax.fori_loop(0, CH, body, 0)
    plsc.subcore_barrier()
    for j in range(RC):
        pltpu.sync_copy(acc_sh.at[pl.ds(sid * RT + j * K, K)], ones_v)
        pltpu.sync_copy(
            ones_v, out_hbm.at[pl.ds(cid * R + sid * RT + j * K, K)]
        )


@functools.partial(
    pl.kernel,
    out_type=jax.ShapeDtypeStruct((NC * R, D), jnp.float32),
    mesh=_mesh,
    scratch_types=[
        pltpu.VMEM((CH, K), jnp.int32),
        pltpu.VMEM((K,), jnp.int32),
        pltpu.VMEM((K,), jnp.int32),
        pltpu.VMEM((K, D), jnp.float32),
        pltpu.VMEM((K, D), jnp.float32),
        pltpu.MemorySpace.VMEM_SHARED((R, D), jnp.float32),
        pltpu.SemaphoreType.DMA,
        pltpu.SemaphoreType.DMA,
        pltpu.SemaphoreType.DMA,
        pltpu.SemaphoreType.DMA,
    ],
)
def _scatter_kernel(g_hbm, src_hbm, dst_hbm, out_hbm, sv, da, db, ra, rb,
                    acc_sh, sem_a, sem_b, sem_da, sem_db):
    cid = lax.axis_index("c")
    sid = lax.axis_index("s")
    wid = cid * NS + sid

    def zfill(i, _):
        for j in range(D // L):
            ra[i, pl.ds(j * L, L)] = jnp.zeros((L,), jnp.float32)
        return 0

    lax.fori_loop(0, K, zfill, 0)
    for j in range(RC):
        pltpu.sync_copy(ra, acc_sh.at[pl.ds(sid * RT + j * K, K)])
    plsc.subcore_barrier()

    pltpu.sync_copy(src_hbm.at[pl.ds(wid * CH, CH)], sv)

    pltpu.async_copy(g_hbm.at[sv.at[0]], ra, sem_a)
    pltpu.async_copy(dst_hbm.at[pl.ds(wid * T, K)], da, sem_da)

    def pair(p, _):
        base = wid * T
        pltpu.async_copy(g_hbm.at[sv.at[2 * p + 1]], rb, sem_b)
        pltpu.async_copy(dst_hbm.at[pl.ds(base + (2 * p + 1) * K, K)], db,
                         sem_db)
        pltpu.make_async_copy(g_hbm.at[pl.ds(0, K)], ra, sem_a).wait()
        pltpu.make_async_copy(dst_hbm.at[pl.ds(0, K)], da, sem_da).wait()
        pltpu.sync_copy(ra, acc_sh.at[da], add=True)
        pltpu.async_copy(g_hbm.at[sv.at[2 * p + 2]], ra, sem_a)
        pltpu.async_copy(dst_hbm.at[pl.ds(base + (2 * p + 2) * K, K)], da,
                         sem_da)
        pltpu.make_async_copy(g_hbm.at[pl.ds(0, K)], rb, sem_b).wait()
        pltpu.make_async_copy(dst_hbm.at[pl.ds(0, K)], db, sem_db).wait()
        pltpu.sync_copy(rb, acc_sh.at[db], add=True)
        return 0

    lax.fori_loop(0, (CH - 2) // 2, pair, 0)
    pltpu.async_copy(g_hbm.at[sv.at[CH - 1]], rb, sem_b)
    pltpu.async_copy(dst_hbm.at[pl.ds(wid * T + (CH - 1) * K, K)], db, sem_db)
    pltpu.make_async_copy(g_hbm.at[pl.ds(0, K)], ra, sem_a).wait()
    pltpu.make_async_copy(dst_hbm.at[pl.ds(0, K)], da, sem_da).wait()
    pltpu.sync_copy(ra, acc_sh.at[da], add=True)
    pltpu.make_async_copy(g_hbm.at[pl.ds(0, K)], rb, sem_b).wait()
    pltpu.make_async_copy(dst_hbm.at[pl.ds(0, K)], db, sem_db).wait()
    pltpu.sync_copy(rb, acc_sh.at[db], add=True)

    plsc.subcore_barrier()
    for j in range(RC):
        pltpu.sync_copy(acc_sh.at[pl.ds(sid * RT + j * K, K)], ra)
        pltpu.sync_copy(
            ra, out_hbm.at[pl.ds(cid * R + sid * RT + j * K, K)]
        )


def _dinv_from(dp):
    deg = dp[0, :, 0] + dp[1, :, 0] + 1.0
    return lax.rsqrt(deg)


def _mm_scale_body(x_ref, w_ref, dp_ref, g_ref):
    dinv = _dinv_from(dp_ref[...])
    h = jnp.dot(x_ref[...], w_ref[...], preferred_element_type=jnp.float32)
    g_ref[...] = h * dinv[:, None]


def _comb_mm_body(p_ref, g_ref, dp_ref, b_ref, w_ref, o_ref):
    dinv = _dinv_from(dp_ref[...])
    p = p_ref[...]
    s = (p[0] + p[1] + g_ref[...]) * dinv[:, None] + b_ref[...][0:1, :]
    a = jnp.maximum(s, 0.0)
    o_ref[...] = (
        jnp.dot(a, w_ref[...], preferred_element_type=jnp.float32)
        * dinv[:, None]
    )


def _final_body(p_ref, g_ref, dp_ref, b_ref, o_ref):
    dinv = _dinv_from(dp_ref[...])
    p = p_ref[...]
    o_ref[...] = (p[0] + p[1] + g_ref[...]) * dinv[:, None] + b_ref[...][0:1, :]


_row_spec = pl.BlockSpec((BM, D), lambda i: (i, 0))
_w_spec = pl.BlockSpec((D, D), lambda i: (0, 0))
_p_spec = pl.BlockSpec((NC, BM, D), lambda i: (0, i, 0))
_b_spec = pl.BlockSpec((8, D), lambda i: (0, 0))
_row_out = jax.ShapeDtypeStruct((N, D), jnp.float32)

_mm_scale = pl.pallas_call(
    _mm_scale_body,
    grid=(GRID,),
    in_specs=[_row_spec, _w_spec, _p_spec],
    out_specs=_row_spec,
    out_shape=_row_out,
)

_comb_mm = pl.pallas_call(
    _comb_mm_body,
    grid=(GRID,),
    in_specs=[_p_spec, _row_spec, _p_spec, _b_spec, _w_spec],
    out_specs=_row_spec,
    out_shape=_row_out,
)

_final = pl.pallas_call(
    _final_body,
    grid=(GRID,),
    in_specs=[_p_spec, _row_spec, _p_spec, _b_spec],
    out_specs=_row_spec,
    out_shape=_row_out,
)


def kernel(x, edge_index, W1, b1, W2, b2):
    ei = edge_index.astype(jnp.int32)
    src, dst = ei[0], ei[1]
    ar = jnp.arange(PAD, dtype=jnp.int32)
    src_p = jnp.concatenate([src, (ar * 37) % N]).reshape(NW * CH, K)
    dst_p = jnp.concatenate([dst, N + ar % (R - N)])
    dst_p2 = dst_p.reshape(NW * CH, K)
    b1r = jnp.broadcast_to(b1.reshape(1, D), (8, D))
    b2r = jnp.broadcast_to(b2.reshape(1, D), (8, D))

    degp = _deg_kernel(dst_p2).reshape(NC, R, D)

    g1 = _mm_scale(x, W1, degp)
    p1 = _scatter_kernel(g1, src_p, dst_p).reshape(NC, R, D)
    g2 = _comb_mm(p1, g1, degp, b1r, W2)
    p2 = _scatter_kernel(g2, src_p, dst_p).reshape(NC, R, D)
    return _final(p2, g2, degp, b2r)


# speedup vs baseline: 29.0846x; 1.0173x over previous
import functools

import jax
import jax.numpy as jnp
from jax import lax
from jax.experimental import pallas as pl
from jax.experimental.pallas import tpu as pltpu
from jax.experimental.pallas import tpu_sc as plsc

N = 10000
E = 320000
D = 128

NC = 2
NS = 16
NW = NC * NS
L = 16

K = 128
CH = 80
T = K * CH
EP = NW * T
PAD = EP - E

R = 10240
RT = R // NS
RC = RT // K

BM = 1000
GRID = N // BM

_mesh = plsc.VectorSubcoreMesh(
    core_axis_name="c", subcore_axis_name="s", num_cores=NC, num_subcores=NS
)


@functools.partial(
    pl.kernel,
    out_type=jax.ShapeDtypeStruct((NC * R, D), jnp.float32),
    mesh=_mesh,
    scratch_types=[
        pltpu.VMEM((CH, K), jnp.int32),
        pltpu.VMEM((K, D), jnp.float32),
        pltpu.MemorySpace.VMEM_SHARED((R, D), jnp.float32),
    ],
)
def _deg_kernel(dst_hbm, out_hbm, dv, ones_v, acc_sh):
    cid = lax.axis_index("c")
    sid = lax.axis_index("s")
    wid = cid * NS + sid

    def zfill(i, _):
        for j in range(D // L):
            ones_v[i, pl.ds(j * L, L)] = jnp.zeros((L,), jnp.float32)
        return 0

    lax.fori_loop(0, K, zfill, 0)
    for j in range(RC):
        pltpu.sync_copy(ones_v, acc_sh.at[pl.ds(sid * RT + j * K, K)])
    plsc.subcore_barrier()

    def ofill(i, _):
        for j in range(D // L):
            ones_v[i, pl.ds(j * L, L)] = jnp.ones((L,), jnp.float32)
        return 0

    lax.fori_loop(0, K, ofill, 0)
    pltpu.sync_copy(dst_hbm.at[pl.ds(wid * CH, CH)], dv)

    def body(ch, _):
        pltpu.sync_copy(ones_v, acc_sh.at[dv.at[ch]], add=True)
        return 0

    lax.fori_loop(0, CH, body, 0)
    plsc.subcore_barrier()
    for j in range(RC):
        pltpu.sync_copy(acc_sh.at[pl.ds(sid * RT + j * K, K)], ones_v)
        pltpu.sync_copy(
            ones_v, out_hbm.at[pl.ds(cid * R + sid * RT + j * K, K)]
        )


@functools.partial(
    pl.kernel,
    out_type=jax.ShapeDtypeStruct((NC * R, D), jnp.float32),
    mesh=_mesh,
    scratch_types=[
        pltpu.VMEM((CH, K), jnp.int32),
        pltpu.VMEM((K,), jnp.int32),
        pltpu.VMEM((K,), jnp.int32),
        pltpu.VMEM((K, D), jnp.float32),
        pltpu.VMEM((K, D), jnp.float32),
        pltpu.MemorySpace.VMEM_SHARED((R, D), jnp.float32),
        pltpu.SemaphoreType.DMA,
        pltpu.SemaphoreType.DMA,
        pltpu.SemaphoreType.DMA,
        pltpu.SemaphoreType.DMA,
    ],
)
def _scatter_kernel(g_hbm, src_hbm, dst_hbm, out_hbm, sv, da, db, ra, rb,
                    acc_sh, sem_a, sem_b, sem_da, sem_db):
    cid = lax.axis_index("c")
    sid = lax.axis_index("s")
    wid = cid * NS + sid

    def zfill(i, _):
        for j in range(D // L):
            rb[i, pl.ds(j * L, L)] = jnp.zeros((L,), jnp.float32)
        return 0

    lax.fori_loop(0, K, zfill, 0)
    pltpu.sync_copy(src_hbm.at[pl.ds(wid * CH, CH)], sv)
    pltpu.async_copy(g_hbm.at[sv.at[0]], ra, sem_a)
    pltpu.async_copy(dst_hbm.at[pl.ds(wid * T, K)], da, sem_da)
    for j in range(RC):
        pltpu.async_copy(rb, acc_sh.at[pl.ds(sid * RT + j * K, K)], sem_db)
    for j in range(RC):
        pltpu.make_async_copy(rb, acc_sh.at[pl.ds(0, K)], sem_db).wait()
    plsc.subcore_barrier()


    def pair(p, _):
        base = wid * T
        pltpu.async_copy(g_hbm.at[sv.at[2 * p + 1]], rb, sem_b)
        pltpu.async_copy(dst_hbm.at[pl.ds(base + (2 * p + 1) * K, K)], db,
                         sem_db)
        pltpu.make_async_copy(g_hbm.at[pl.ds(0, K)], ra, sem_a).wait()
        pltpu.make_async_copy(dst_hbm.at[pl.ds(0, K)], da, sem_da).wait()
        pltpu.sync_copy(ra, acc_sh.at[da], add=True)
        pltpu.async_copy(g_hbm.at[sv.at[2 * p + 2]], ra, sem_a)
        pltpu.async_copy(dst_hbm.at[pl.ds(base + (2 * p + 2) * K, K)], da,
                         sem_da)
        pltpu.make_async_copy(g_hbm.at[pl.ds(0, K)], rb, sem_b).wait()
        pltpu.make_async_copy(dst_hbm.at[pl.ds(0, K)], db, sem_db).wait()
        pltpu.sync_copy(rb, acc_sh.at[db], add=True)
        return 0

    lax.fori_loop(0, (CH - 2) // 2, pair, 0)
    pltpu.async_copy(g_hbm.at[sv.at[CH - 1]], rb, sem_b)
    pltpu.async_copy(dst_hbm.at[pl.ds(wid * T + (CH - 1) * K, K)], db, sem_db)
    pltpu.make_async_copy(g_hbm.at[pl.ds(0, K)], ra, sem_a).wait()
    pltpu.make_async_copy(dst_hbm.at[pl.ds(0, K)], da, sem_da).wait()
    pltpu.sync_copy(ra, acc_sh.at[da], add=True)
    pltpu.make_async_copy(g_hbm.at[pl.ds(0, K)], rb, sem_b).wait()
    pltpu.make_async_copy(dst_hbm.at[pl.ds(0, K)], db, sem_db).wait()
    pltpu.sync_copy(rb, acc_sh.at[db], add=True)

    plsc.subcore_barrier()
    bufs = (ra, rb)
    rsems = (sem_a, sem_b)
    wsems = (sem_da, sem_db)
    pltpu.async_copy(acc_sh.at[pl.ds(sid * RT, K)], ra, sem_a)
    for j in range(RC):
        b = bufs[j % 2]
        pltpu.make_async_copy(acc_sh.at[pl.ds(0, K)], b, rsems[j % 2]).wait()
        pltpu.async_copy(
            b, out_hbm.at[pl.ds(cid * R + sid * RT + j * K, K)], wsems[j % 2]
        )
        if j + 1 < RC:
            nb = bufs[(j + 1) % 2]
            if j >= 1:
                pltpu.make_async_copy(
                    nb, out_hbm.at[pl.ds(0, K)], wsems[(j + 1) % 2]
                ).wait()
            pltpu.async_copy(
                acc_sh.at[pl.ds(sid * RT + (j + 1) * K, K)], nb,
                rsems[(j + 1) % 2]
            )
    pltpu.make_async_copy(
        bufs[(RC - 2) % 2], out_hbm.at[pl.ds(0, K)], wsems[(RC - 2) % 2]
    ).wait()
    pltpu.make_async_copy(
        bufs[(RC - 1) % 2], out_hbm.at[pl.ds(0, K)], wsems[(RC - 1) % 2]
    ).wait()


def _dinv_from(dp):
    deg = dp[0, :, 0] + dp[1, :, 0] + 1.0
    return lax.rsqrt(deg)


def _mm_scale_body(x_ref, w_ref, dp_ref, g_ref):
    dinv = _dinv_from(dp_ref[...])
    h = jnp.dot(x_ref[...], w_ref[...], preferred_element_type=jnp.float32)
    g_ref[...] = h * dinv[:, None]


def _comb_mm_body(p_ref, g_ref, dp_ref, b_ref, w_ref, o_ref):
    dinv = _dinv_from(dp_ref[...])
    p = p_ref[...]
    s = (p[0] + p[1] + g_ref[...]) * dinv[:, None] + b_ref[...][0:1, :]
    a = jnp.maximum(s, 0.0)
    o_ref[...] = (
        jnp.dot(a, w_ref[...], preferred_element_type=jnp.float32)
        * dinv[:, None]
    )


def _final_body(p_ref, g_ref, dp_ref, b_ref, o_ref):
    dinv = _dinv_from(dp_ref[...])
    p = p_ref[...]
    o_ref[...] = (p[0] + p[1] + g_ref[...]) * dinv[:, None] + b_ref[...][0:1, :]


_row_spec = pl.BlockSpec((BM, D), lambda i: (i, 0))
_w_spec = pl.BlockSpec((D, D), lambda i: (0, 0))
_p_spec = pl.BlockSpec((NC, BM, D), lambda i: (0, i, 0))
_b_spec = pl.BlockSpec((8, D), lambda i: (0, 0))
_row_out = jax.ShapeDtypeStruct((N, D), jnp.float32)

_mm_scale = pl.pallas_call(
    _mm_scale_body,
    grid=(GRID,),
    in_specs=[_row_spec, _w_spec, _p_spec],
    out_specs=_row_spec,
    out_shape=_row_out,
)

_comb_mm = pl.pallas_call(
    _comb_mm_body,
    grid=(GRID,),
    in_specs=[_p_spec, _row_spec, _p_spec, _b_spec, _w_spec],
    out_specs=_row_spec,
    out_shape=_row_out,
)

_final = pl.pallas_call(
    _final_body,
    grid=(GRID,),
    in_specs=[_p_spec, _row_spec, _p_spec, _b_spec],
    out_specs=_row_spec,
    out_shape=_row_out,
)


def kernel(x, edge_index, W1, b1, W2, b2):
    ei = edge_index.astype(jnp.int32)
    src, dst = ei[0], ei[1]
    ar = jnp.arange(PAD, dtype=jnp.int32)
    src_p = jnp.concatenate([src, (ar * 37) % N]).reshape(NW * CH, K)
    dst_p = jnp.concatenate([dst, N + ar % (R - N)])
    dst_p2 = dst_p.reshape(NW * CH, K)
    b1r = jnp.broadcast_to(b1.reshape(1, D), (8, D))
    b2r = jnp.broadcast_to(b2.reshape(1, D), (8, D))

    degp = _deg_kernel(dst_p2).reshape(NC, R, D)

    g1 = _mm_scale(x, W1, degp)
    p1 = _scatter_kernel(g1, src_p, dst_p).reshape(NC, R, D)
    g2 = _comb_mm(p1, g1, degp, b1r, W2)
    p2 = _scatter_kernel(g2, src_p, dst_p).reshape(NC, R, D)
    return _final(p2, g2, degp, b2r)


# speedup vs baseline: 29.2295x; 1.0050x over previous
import functools

import jax
import jax.numpy as jnp
from jax import lax
from jax.experimental import pallas as pl
from jax.experimental.pallas import tpu as pltpu
from jax.experimental.pallas import tpu_sc as plsc

N = 10000
E = 320000
D = 128

NC = 2
NS = 16
NW = NC * NS
L = 16

K = 128
CH = 80
T = K * CH
EP = NW * T
PAD = EP - E

R = 10240
RT = R // NS
RC = RT // K

BM = 1000
GRID = N // BM

_mesh = plsc.VectorSubcoreMesh(
    core_axis_name="c", subcore_axis_name="s", num_cores=NC, num_subcores=NS
)


@functools.partial(
    pl.kernel,
    out_type=jax.ShapeDtypeStruct((NC * R, D), jnp.float32),
    mesh=_mesh,
    scratch_types=[
        pltpu.VMEM((CH, K), jnp.int32),
        pltpu.VMEM((K, D), jnp.float32),
        pltpu.VMEM((K, D), jnp.float32),
        pltpu.MemorySpace.VMEM_SHARED((R, D), jnp.float32),
        pltpu.SemaphoreType.DMA,
        pltpu.SemaphoreType.DMA,
        pltpu.SemaphoreType.DMA,
        pltpu.SemaphoreType.DMA,
    ],
)
def _deg_kernel(dst_hbm, out_hbm, dv, ones_v, zb, acc_sh, sem_a, sem_b,
                sem_c, sem_d):
    cid = lax.axis_index("c")
    sid = lax.axis_index("s")
    wid = cid * NS + sid

    def zfill(i, _):
        for j in range(D // L):
            zb[i, pl.ds(j * L, L)] = jnp.zeros((L,), jnp.float32)
        return 0

    lax.fori_loop(0, K, zfill, 0)
    pltpu.sync_copy(dst_hbm.at[pl.ds(wid * CH, CH)], dv)
    for j in range(RC):
        pltpu.async_copy(zb, acc_sh.at[pl.ds(sid * RT + j * K, K)], sem_a)

    def ofill(i, _):
        for j in range(D // L):
            ones_v[i, pl.ds(j * L, L)] = jnp.ones((L,), jnp.float32)
        return 0

    lax.fori_loop(0, K, ofill, 0)
    for j in range(RC):
        pltpu.make_async_copy(zb, acc_sh.at[pl.ds(0, K)], sem_a).wait()
    plsc.subcore_barrier()

    def body(ch, _):
        pltpu.sync_copy(ones_v, acc_sh.at[dv.at[ch]], add=True)
        return 0

    lax.fori_loop(0, CH, body, 0)
    plsc.subcore_barrier()
    bufs = (ones_v, zb)
    rsems = (sem_a, sem_b)
    wsems = (sem_c, sem_d)
    pltpu.async_copy(acc_sh.at[pl.ds(sid * RT, K)], ones_v, sem_a)
    for j in range(RC):
        b = bufs[j % 2]
        pltpu.make_async_copy(acc_sh.at[pl.ds(0, K)], b, rsems[j % 2]).wait()
        pltpu.async_copy(
            b, out_hbm.at[pl.ds(cid * R + sid * RT + j * K, K)], wsems[j % 2]
        )
        if j + 1 < RC:
            nb = bufs[(j + 1) % 2]
            if j >= 1:
                pltpu.make_async_copy(
                    nb, out_hbm.at[pl.ds(0, K)], wsems[(j + 1) % 2]
                ).wait()
            pltpu.async_copy(
                acc_sh.at[pl.ds(sid * RT + (j + 1) * K, K)], nb,
                rsems[(j + 1) % 2]
            )
    pltpu.make_async_copy(
        bufs[(RC - 2) % 2], out_hbm.at[pl.ds(0, K)], wsems[(RC - 2) % 2]
    ).wait()
    pltpu.make_async_copy(
        bufs[(RC - 1) % 2], out_hbm.at[pl.ds(0, K)], wsems[(RC - 1) % 2]
    ).wait()


@functools.partial(
    pl.kernel,
    out_type=jax.ShapeDtypeStruct((NC * R, D), jnp.float32),
    mesh=_mesh,
    scratch_types=[
        pltpu.VMEM((CH, K), jnp.int32),
        pltpu.VMEM((K,), jnp.int32),
        pltpu.VMEM((K,), jnp.int32),
        pltpu.VMEM((K, D), jnp.float32),
        pltpu.VMEM((K, D), jnp.float32),
        pltpu.MemorySpace.VMEM_SHARED((R, D), jnp.float32),
        pltpu.SemaphoreType.DMA,
        pltpu.SemaphoreType.DMA,
        pltpu.SemaphoreType.DMA,
        pltpu.SemaphoreType.DMA,
    ],
)
def _scatter_kernel(g_hbm, src_hbm, dst_hbm, out_hbm, sv, da, db, ra, rb,
                    acc_sh, sem_a, sem_b, sem_da, sem_db):
    cid = lax.axis_index("c")
    sid = lax.axis_index("s")
    wid = cid * NS + sid

    def zfill(i, _):
        for j in range(D // L):
            rb[i, pl.ds(j * L, L)] = jnp.zeros((L,), jnp.float32)
        return 0

    lax.fori_loop(0, K, zfill, 0)
    pltpu.sync_copy(src_hbm.at[pl.ds(wid * CH, CH)], sv)
    pltpu.async_copy(g_hbm.at[sv.at[0]], ra, sem_a)
    pltpu.async_copy(dst_hbm.at[pl.ds(wid * T, K)], da, sem_da)
    for j in range(RC):
        pltpu.async_copy(rb, acc_sh.at[pl.ds(sid * RT + j * K, K)], sem_db)
    for j in range(RC):
        pltpu.make_async_copy(rb, acc_sh.at[pl.ds(0, K)], sem_db).wait()
    plsc.subcore_barrier()


    def pair(p, _):
        base = wid * T
        pltpu.async_copy(g_hbm.at[sv.at[2 * p + 1]], rb, sem_b)
        pltpu.async_copy(dst_hbm.at[pl.ds(base + (2 * p + 1) * K, K)], db,
                         sem_db)
        pltpu.make_async_copy(g_hbm.at[pl.ds(0, K)], ra, sem_a).wait()
        pltpu.make_async_copy(dst_hbm.at[pl.ds(0, K)], da, sem_da).wait()
        pltpu.sync_copy(ra, acc_sh.at[da], add=True)
        pltpu.async_copy(g_hbm.at[sv.at[2 * p + 2]], ra, sem_a)
        pltpu.async_copy(dst_hbm.at[pl.ds(base + (2 * p + 2) * K, K)], da,
                         sem_da)
        pltpu.make_async_copy(g_hbm.at[pl.ds(0, K)], rb, sem_b).wait()
        pltpu.make_async_copy(dst_hbm.at[pl.ds(0, K)], db, sem_db).wait()
        pltpu.sync_copy(rb, acc_sh.at[db], add=True)
        return 0

    lax.fori_loop(0, (CH - 2) // 2, pair, 0)
    pltpu.async_copy(g_hbm.at[sv.at[CH - 1]], rb, sem_b)
    pltpu.async_copy(dst_hbm.at[pl.ds(wid * T + (CH - 1) * K, K)], db, sem_db)
    pltpu.make_async_copy(g_hbm.at[pl.ds(0, K)], ra, sem_a).wait()
    pltpu.make_async_copy(dst_hbm.at[pl.ds(0, K)], da, sem_da).wait()
    pltpu.sync_copy(ra, acc_sh.at[da], add=True)
    pltpu.make_async_copy(g_hbm.at[pl.ds(0, K)], rb, sem_b).wait()
    pltpu.make_async_copy(dst_hbm.at[pl.ds(0, K)], db, sem_db).wait()
    pltpu.sync_copy(rb, acc_sh.at[db], add=True)

    plsc.subcore_barrier()
    bufs = (ra, rb)
    rsems = (sem_a, sem_b)
    wsems = (sem_da, sem_db)
    pltpu.async_copy(acc_sh.at[pl.ds(sid * RT, K)], ra, sem_a)
    for j in range(RC):
        b = bufs[j % 2]
        pltpu.make_async_copy(acc_sh.at[pl.ds(0, K)], b, rsems[j % 2]).wait()
        pltpu.async_copy(
            b, out_hbm.at[pl.ds(cid * R + sid * RT + j * K, K)], wsems[j % 2]
        )
        if j + 1 < RC:
            nb = bufs[(j + 1) % 2]
            if j >= 1:
                pltpu.make_async_copy(
                    nb, out_hbm.at[pl.ds(0, K)], wsems[(j + 1) % 2]
                ).wait()
            pltpu.async_copy(
                acc_sh.at[pl.ds(sid * RT + (j + 1) * K, K)], nb,
                rsems[(j + 1) % 2]
            )
    pltpu.make_async_copy(
        bufs[(RC - 2) % 2], out_hbm.at[pl.ds(0, K)], wsems[(RC - 2) % 2]
    ).wait()
    pltpu.make_async_copy(
        bufs[(RC - 1) % 2], out_hbm.at[pl.ds(0, K)], wsems[(RC - 1) % 2]
    ).wait()


def _dinv_from(dp):
    deg = dp[0, :, 0] + dp[1, :, 0] + 1.0
    return lax.rsqrt(deg)


def _mm_scale_body(x_ref, w_ref, dp_ref, g_ref):
    dinv = _dinv_from(dp_ref[...])
    h = jnp.dot(x_ref[...], w_ref[...], preferred_element_type=jnp.float32)
    g_ref[...] = h * dinv[:, None]


def _comb_mm_body(p_ref, g_ref, dp_ref, b_ref, w_ref, o_ref):
    dinv = _dinv_from(dp_ref[...])
    p = p_ref[...]
    s = (p[0] + p[1] + g_ref[...]) * dinv[:, None] + b_ref[...][0:1, :]
    a = jnp.maximum(s, 0.0)
    o_ref[...] = (
        jnp.dot(a, w_ref[...], preferred_element_type=jnp.float32)
        * dinv[:, None]
    )


def _final_body(p_ref, g_ref, dp_ref, b_ref, o_ref):
    dinv = _dinv_from(dp_ref[...])
    p = p_ref[...]
    o_ref[...] = (p[0] + p[1] + g_ref[...]) * dinv[:, None] + b_ref[...][0:1, :]


_row_spec = pl.BlockSpec((BM, D), lambda i: (i, 0))
_w_spec = pl.BlockSpec((D, D), lambda i: (0, 0))
_p_spec = pl.BlockSpec((NC, BM, D), lambda i: (0, i, 0))
_b_spec = pl.BlockSpec((8, D), lambda i: (0, 0))
_row_out = jax.ShapeDtypeStruct((N, D), jnp.float32)

_mm_scale = pl.pallas_call(
    _mm_scale_body,
    grid=(GRID,),
    in_specs=[_row_spec, _w_spec, _p_spec],
    out_specs=_row_spec,
    out_shape=_row_out,
)

_comb_mm = pl.pallas_call(
    _comb_mm_body,
    grid=(GRID,),
    in_specs=[_p_spec, _row_spec, _p_spec, _b_spec, _w_spec],
    out_specs=_row_spec,
    out_shape=_row_out,
)

_final = pl.pallas_call(
    _final_body,
    grid=(GRID,),
    in_specs=[_p_spec, _row_spec, _p_spec, _b_spec],
    out_specs=_row_spec,
    out_shape=_row_out,
)


def kernel(x, edge_index, W1, b1, W2, b2):
    ei = edge_index.astype(jnp.int32)
    src, dst = ei[0], ei[1]
    ar = jnp.arange(PAD, dtype=jnp.int32)
    src_p = jnp.concatenate([src, (ar * 37) % N]).reshape(NW * CH, K)
    dst_p = jnp.concatenate([dst, N + ar % (R - N)])
    dst_p2 = dst_p.reshape(NW * CH, K)
    b1r = jnp.broadcast_to(b1.reshape(1, D), (8, D))
    b2r = jnp.broadcast_to(b2.reshape(1, D), (8, D))

    degp = _deg_kernel(dst_p2).reshape(NC, R, D)

    g1 = _mm_scale(x, W1, degp)
    p1 = _scatter_kernel(g1, src_p, dst_p).reshape(NC, R, D)
    g2 = _comb_mm(p1, g1, degp, b1r, W2)
    p2 = _scatter_kernel(g2, src_p, dst_p).reshape(NC, R, D)
    return _final(p2, g2, degp, b2r)
